# baseline (speedup 1.0000x reference)
import jax
import jax.numpy as jnp
from jax import lax
from jax.experimental import pallas as pl
from jax.experimental.pallas import tpu as pltpu
from jax.experimental.pallas import tpu_sc as plsc

B = 4096
L = 50
D = 64
V = 1000002
BL = B * L
NC = 2
NS = 16
NW = NC * NS
ROWS_PER_W = BL // NW
GSZ = 100
CHUNK = 2 * GSZ
BPC = CHUNK // L
CPW = ROWS_PER_W // CHUNK
IPW = ROWS_PER_W // GSZ
NBUF = 2
NGRP = CPW // NBUF


def _sc_body(left_hbm, table_hbm, pos4_hbm, out_hbm, idx_all, rows, pos4,
             gsem, ssem):
    wid = lax.axis_index("s") * NC + lax.axis_index("c")
    bat0 = wid * (ROWS_PER_W // L)

    pltpu.sync_copy(left_hbm.at[pl.ds(wid * IPW, IPW)], idx_all)
    pltpu.sync_copy(pos4_hbm, pos4)

    @pl.loop(0, NGRP)
    def _group(g):
        descs = []
        for b in range(NBUF):
            c = g * NBUF + b

            @pl.when(g > 0)
            def _():
                for k in range(BPC):
                    pltpu.make_async_copy(
                        rows.at[b, pl.ds(k * L, L), pl.ds(0, D)],
                        out_hbm.at[bat0 + k],
                        ssem.at[b],
                    ).wait()

            descs.append((
                pltpu.async_copy(table_hbm.at[idx_all.at[2 * c]],
                                 rows.at[b, pl.ds(0, GSZ)], gsem.at[b]),
                pltpu.async_copy(table_hbm.at[idx_all.at[2 * c + 1]],
                                 rows.at[b, pl.ds(GSZ, GSZ)], gsem.at[b]),
            ))

        for b in range(NBUF):
            c = g * NBUF + b
            for d in descs[b]:
                d.wait()

            @pl.loop(0, CHUNK)
            def _row(r):
                for q in range(D // 16):
                    x = pos4[r, pl.ds(q * 16, 16)]
                    plsc.addupdate(rows.at[b, r, pl.ds(q * 16, 16)], x)

            for k in range(BPC):
                pltpu.async_copy(rows.at[b, pl.ds(k * L, L), pl.ds(0, D)],
                                 out_hbm.at[c * BPC + bat0 + k], ssem.at[b])

    for b in range(NBUF):
        for k in range(BPC):
            pltpu.make_async_copy(rows.at[b, pl.ds(k * L, L), pl.ds(0, D)],
                                  out_hbm.at[bat0 + k], ssem.at[b]).wait()


@jax.jit
def _sc_gather(left2d, emb_left, pos4):
    mesh = plsc.VectorSubcoreMesh(core_axis_name="c", subcore_axis_name="s",
                                  num_cores=NC, num_subcores=NS)
    return pl.kernel(
        _sc_body,
        out_type=jax.ShapeDtypeStruct((B, L, D), jnp.float32),
        mesh=mesh,
        compiler_params=pltpu.CompilerParams(use_tc_tiling_on_sc=False),
        scratch_types=[
            pltpu.VMEM((IPW, GSZ), jnp.int32),
            pltpu.VMEM((NBUF, CHUNK, 2 * D), jnp.float32),
            pltpu.VMEM((CHUNK, D), jnp.float32),
            pltpu.SemaphoreType.DMA((NBUF,)),
            pltpu.SemaphoreType.DMA((NBUF,)),
        ],
    )(left2d, emb_left, pos4)


def _mask_body(len_ref, out_ref):
    lens = len_ref[...]
    iota = lax.broadcasted_iota(jnp.int32, (B, L), 1)
    out_ref[...] = iota < lens


@jax.jit
def _mask_call(length):
    return pl.pallas_call(
        _mask_body,
        out_shape=jax.ShapeDtypeStruct((B, L), jnp.bool_),
    )(length)


def kernel(left, length, emb_left, pos_emb):
    left2d = left.reshape(BL // GSZ, GSZ)
    pos4 = jnp.tile(pos_emb[:L], (CHUNK // L, 1))
    table_pad = jnp.pad(emb_left, ((0, 8 - V % 8), (0, 2 * D - D)))
    seq = _sc_gather(left2d, table_pad, pos4)
    mask = _mask_call(length)
    return seq, mask


# speedup vs baseline: 1.1606x; 1.1606x over previous
import jax
import jax.numpy as jnp
from jax import lax
from jax.experimental import pallas as pl
from jax.experimental.pallas import tpu as pltpu
from jax.experimental.pallas import tpu_sc as plsc

B = 4096
L = 50
D = 64
V = 1000002
V = 1000002
BL = B * L
NC = 2
NS = 16
NW = NC * NS
ROWS_PER_W = BL // NW
GSZ = 100
CHUNK = 2 * GSZ
BPC = CHUNK // L
CPW = ROWS_PER_W // CHUNK
IPW = ROWS_PER_W // GSZ
NBUF = 4
NGRP = CPW // NBUF


def _sc_body(left_hbm, table_hbm, pos4_hbm, out_hbm, idx_all, rows, pos4,
             gsem, ssem):
    wid = lax.axis_index("s") * NC + lax.axis_index("c")
    bat0 = wid * (ROWS_PER_W // L)

    pltpu.sync_copy(left_hbm.at[pl.ds(wid * IPW, IPW)], idx_all)
    pltpu.sync_copy(pos4_hbm, pos4)

    @pl.loop(0, NGRP)
    def _group(g):
        descs = []
        for b in range(NBUF):
            c = g * NBUF + b

            @pl.when(g > 0)
            def _():
                for k in range(BPC):
                    pltpu.make_async_copy(
                        rows.at[b, pl.ds(k * L, L)],
                        out_hbm.at[bat0 + k],
                        ssem.at[b],
                    ).wait()

            descs.append((
                pltpu.async_copy(table_hbm.at[idx_all.at[2 * c]],
                                 rows.at[b, pl.ds(0, GSZ)], gsem.at[b]),
                pltpu.async_copy(table_hbm.at[idx_all.at[2 * c + 1]],
                                 rows.at[b, pl.ds(GSZ, GSZ)], gsem.at[b]),
            ))

        for b in range(NBUF):
            c = g * NBUF + b
            for d in descs[b]:
                d.wait()

            @pl.loop(0, CHUNK)
            def _row(r):
                for q in range(D // 16):
                    x = pos4[r, pl.ds(q * 16, 16)]
                    plsc.addupdate(rows.at[b, r, pl.ds(q * 16, 16)], x)

            for k in range(BPC):
                pltpu.async_copy(rows.at[b, pl.ds(k * L, L)],
                                 out_hbm.at[c * BPC + bat0 + k], ssem.at[b])

    for b in range(NBUF):
        for k in range(BPC):
            pltpu.make_async_copy(rows.at[b, pl.ds(k * L, L)],
                                  out_hbm.at[bat0 + k], ssem.at[b]).wait()


@jax.jit
def _sc_gather(left2d, emb_left, pos4):
    mesh = plsc.VectorSubcoreMesh(core_axis_name="c", subcore_axis_name="s",
                                  num_cores=NC, num_subcores=NS)
    return pl.kernel(
        _sc_body,
        out_type=jax.ShapeDtypeStruct((B, L, D), jnp.float32),
        mesh=mesh,
        compiler_params=pltpu.CompilerParams(use_tc_tiling_on_sc=False),
        scratch_types=[
            pltpu.VMEM((IPW, GSZ), jnp.int32),
            pltpu.VMEM((NBUF, CHUNK, D), jnp.float32),
            pltpu.VMEM((CHUNK, D), jnp.float32),
            pltpu.SemaphoreType.DMA((NBUF,)),
            pltpu.SemaphoreType.DMA((NBUF,)),
        ],
    )(left2d, emb_left, pos4)


def _mask_body(len_ref, out_ref):
    lens = len_ref[...]
    iota = lax.broadcasted_iota(jnp.int32, (B, L), 1)
    out_ref[...] = iota < lens


@jax.jit
def _mask_call(length):
    return pl.pallas_call(
        _mask_body,
        out_shape=jax.ShapeDtypeStruct((B, L), jnp.bool_),
    )(length)


def kernel(left, length, emb_left, pos_emb):
    left2d = left.reshape(BL // GSZ, GSZ) * 2
    pos4 = jnp.tile(pos_emb[:L], (CHUNK // L, 1))
    table_pad = jnp.pad(emb_left, ((0, 8 - V % 8), (0, D))).reshape(-1, D)
    seq = _sc_gather(left2d, table_pad, pos4)
    mask = _mask_call(length)
    return seq, mask


# speedup vs baseline: 1.2780x; 1.1012x over previous
import jax
import jax.numpy as jnp
from jax import lax
from jax.experimental import pallas as pl
from jax.experimental.pallas import tpu as pltpu
from jax.experimental.pallas import tpu_sc as plsc

B = 4096
L = 50
D = 64
V = 1000002
V = 1000002
BL = B * L
NC = 2
NS = 16
NW = NC * NS
ROWS_PER_W = BL // NW
GSZ = 100
CHUNK = 2 * GSZ
BPC = CHUNK // L
CPW = ROWS_PER_W // CHUNK
IPW = ROWS_PER_W // GSZ
NBUF = 4
NGRP = CPW // NBUF


def _sc_body(left_hbm, table_hbm, pos4_hbm, out_hbm, idx_all, rows, pos4,
             gsem, ssem):
    wid = lax.axis_index("s") * NC + lax.axis_index("c")
    bat0 = wid * (ROWS_PER_W // L)

    pltpu.sync_copy(left_hbm.at[pl.ds(wid * IPW, IPW)], idx_all)
    pltpu.sync_copy(pos4_hbm, pos4)

    @pl.loop(0, NGRP)
    def _group(g):
        descs = []
        for b in range(NBUF):
            c = g * NBUF + b

            @pl.when(g > 0)
            def _():
                for k in range(BPC):
                    pltpu.make_async_copy(
                        rows.at[b, pl.ds(k * L, L)],
                        out_hbm.at[bat0 + k],
                        ssem.at[b],
                    ).wait()

            descs.append((
                pltpu.async_copy(table_hbm.at[idx_all.at[2 * c]],
                                 rows.at[b, pl.ds(0, GSZ)], gsem.at[b]),
                pltpu.async_copy(table_hbm.at[idx_all.at[2 * c + 1]],
                                 rows.at[b, pl.ds(GSZ, GSZ)], gsem.at[b]),
            ))

        for b in range(NBUF):
            c = g * NBUF + b
            for d in descs[b]:
                d.wait()

            @pl.loop(0, CHUNK)
            def _row(r):
                for q in range(D // 16):
                    x = pos4[r, pl.ds(q * 16, 16)]
                    plsc.addupdate(rows.at[b, r, pl.ds(q * 16, 16)], x)

            for k in range(BPC):
                pltpu.async_copy(rows.at[b, pl.ds(k * L, L)],
                                 out_hbm.at[c * BPC + bat0 + k], ssem.at[b])

    for b in range(NBUF):
        for k in range(BPC):
            pltpu.make_async_copy(rows.at[b, pl.ds(k * L, L)],
                                  out_hbm.at[bat0 + k], ssem.at[b]).wait()


@jax.jit
def _sc_gather(left2d, emb_left, pos4):
    mesh = plsc.VectorSubcoreMesh(core_axis_name="c", subcore_axis_name="s",
                                  num_cores=NC, num_subcores=NS)
    return pl.kernel(
        _sc_body,
        out_type=jax.ShapeDtypeStruct((B, L, D), jnp.float32),
        mesh=mesh,
        compiler_params=pltpu.CompilerParams(use_tc_tiling_on_sc=False),
        scratch_types=[
            pltpu.VMEM((IPW, GSZ), jnp.int32),
            pltpu.VMEM((NBUF, CHUNK, D), jnp.float32),
            pltpu.VMEM((CHUNK, D), jnp.float32),
            pltpu.SemaphoreType.DMA((NBUF,)),
            pltpu.SemaphoreType.DMA((NBUF,)),
        ],
    )(left2d, emb_left, pos4)


BC = 2048
NPAD = V + (8 - V % 8)


def _trans_body(in_ref, out_ref):
    x = in_ref[...]
    out_ref[:, 0:D] = jnp.swapaxes(x, 0, 1)


@jax.jit
def _detile(emb_t):
    grid = (NPAD + BC - 1) // BC
    return pl.pallas_call(
        _trans_body,
        grid=(grid,),
        in_specs=[pl.BlockSpec((64, BC), lambda g: (0, g))],
        out_specs=pl.BlockSpec((BC, 2 * D), lambda g: (g, 0)),
        out_shape=jax.ShapeDtypeStruct((NPAD, 2 * D), jnp.float32),
    )(emb_t)


def _mask_body(len_ref, out_ref):
    lens = len_ref[...]
    iota = lax.broadcasted_iota(jnp.int32, (B, L), 1)
    out_ref[...] = iota < lens


@jax.jit
def _mask_call(length):
    return pl.pallas_call(
        _mask_body,
        out_shape=jax.ShapeDtypeStruct((B, L), jnp.bool_),
    )(length)


def kernel(left, length, emb_left, pos_emb):
    left2d = left.reshape(BL // GSZ, GSZ) * 2
    pos4 = jnp.tile(pos_emb[:L], (CHUNK // L, 1))
    table_pad = _detile(emb_left.T).reshape(-1, D)
    seq = _sc_gather(left2d, table_pad, pos4)
    mask = _mask_call(length)
    return seq, mask


# speedup vs baseline: 1.5721x; 1.2301x over previous
import jax
import jax.numpy as jnp
from jax import lax
from jax.experimental import pallas as pl
from jax.experimental.pallas import tpu as pltpu
from jax.experimental.pallas import tpu_sc as plsc

B = 4096
L = 50
D = 64
V = 1000002
V = 1000002
BL = B * L
NC = 2
NS = 16
NW = NC * NS
ROWS_PER_W = BL // NW
GSZ = 100
CHUNK = 2 * GSZ
BPC = CHUNK // L
CPW = ROWS_PER_W // CHUNK
IPW = ROWS_PER_W // GSZ
NBUF = 4
NGRP = CPW // NBUF


def _sc_body(left_hbm, table_hbm, pos4_hbm, out_hbm, idx_all, rows, pos4,
             gsem, ssem):
    wid = lax.axis_index("s") * NC + lax.axis_index("c")
    bat0 = wid * (ROWS_PER_W // L)

    pltpu.sync_copy(left_hbm.at[pl.ds(wid * IPW, IPW)], idx_all)
    pltpu.sync_copy(pos4_hbm, pos4)

    @pl.loop(0, NGRP)
    def _group(g):
        descs = []
        for b in range(NBUF):
            c = g * NBUF + b

            @pl.when(g > 0)
            def _():
                for k in range(BPC):
                    pltpu.make_async_copy(
                        rows.at[b, pl.ds(k * L, L)],
                        out_hbm.at[bat0 + k],
                        ssem.at[b],
                    ).wait()

            descs.append((
                pltpu.async_copy(table_hbm.at[idx_all.at[2 * c]],
                                 rows.at[b, pl.ds(0, GSZ)], gsem.at[b]),
                pltpu.async_copy(table_hbm.at[idx_all.at[2 * c + 1]],
                                 rows.at[b, pl.ds(GSZ, GSZ)], gsem.at[b]),
            ))

        for b in range(NBUF):
            c = g * NBUF + b
            for d in descs[b]:
                d.wait()

            @pl.loop(0, CHUNK)
            def _row(r):
                for q in range(D // 16):
                    x = pos4[r, pl.ds(q * 16, 16)]
                    plsc.addupdate(rows.at[b, r, pl.ds(q * 16, 16)], x)

            for k in range(BPC):
                pltpu.async_copy(rows.at[b, pl.ds(k * L, L)],
                                 out_hbm.at[c * BPC + bat0 + k], ssem.at[b])

    for b in range(NBUF):
        for k in range(BPC):
            pltpu.make_async_copy(rows.at[b, pl.ds(k * L, L)],
                                  out_hbm.at[bat0 + k], ssem.at[b]).wait()


@jax.jit
def _sc_gather(left2d, emb_left, pos4):
    mesh = plsc.VectorSubcoreMesh(core_axis_name="c", subcore_axis_name="s",
                                  num_cores=NC, num_subcores=NS)
    return pl.kernel(
        _sc_body,
        out_type=jax.ShapeDtypeStruct((B, L, D), jnp.float32),
        mesh=mesh,
        compiler_params=pltpu.CompilerParams(use_tc_tiling_on_sc=False),
        scratch_types=[
            pltpu.VMEM((IPW, GSZ), jnp.int32),
            pltpu.VMEM((NBUF, CHUNK, D), jnp.float32),
            pltpu.VMEM((CHUNK, D), jnp.float32),
            pltpu.SemaphoreType.DMA((NBUF,)),
            pltpu.SemaphoreType.DMA((NBUF,)),
        ],
    )(left2d, emb_left, pos4)


BC = 4096
NPAD = V + (8 - V % 8)


def _trans_body(in_ref, out_ref):
    x = in_ref[...]
    out_ref[:, 0:D] = jnp.swapaxes(x, 0, 1)


@jax.jit
def _detile(emb_t):
    grid = (NPAD + BC - 1) // BC
    return pl.pallas_call(
        _trans_body,
        grid=(grid,),
        in_specs=[pl.BlockSpec((64, BC), lambda g: (0, g))],
        out_specs=pl.BlockSpec((BC, 2 * D), lambda g: (g, 0)),
        out_shape=jax.ShapeDtypeStruct((NPAD, 2 * D), jnp.float32),
    )(emb_t)


def _mask_body(len_ref, out_ref):
    lens = len_ref[...]
    iota = lax.broadcasted_iota(jnp.int32, (B, L), 1)
    out_ref[...] = iota < lens


@jax.jit
def _mask_call(length):
    return pl.pallas_call(
        _mask_body,
        out_shape=jax.ShapeDtypeStruct((B, L), jnp.bool_),
    )(length)


def kernel(left, length, emb_left, pos_emb):
    left2d = left.reshape(BL // GSZ, GSZ) * 2
    pos4 = jnp.tile(pos_emb[:L], (CHUNK // L, 1))
    table_pad = _detile(emb_left.T).reshape(-1, D)
    seq = _sc_gather(left2d, table_pad, pos4)
    mask = _mask_call(length)
    return seq, mask


# speedup vs baseline: 1.8191x; 1.1571x over previous
import jax
import jax.numpy as jnp
from jax import lax
from jax.experimental import pallas as pl
from jax.experimental.pallas import tpu as pltpu
from jax.experimental.pallas import tpu_sc as plsc

B = 4096
L = 50
D = 64
V = 1000002
BL = B * L
NC = 2
NS = 16
NW = NC * NS
ROWS_PER_W = BL // NW
GSZ = 100
CHUNK = 2 * GSZ
BPC = CHUNK // L
CPW = ROWS_PER_W // CHUNK
IPW = ROWS_PER_W // GSZ
NBUF = 4
NGRP = CPW // NBUF


def _sc_body(left_hbm, table_hbm, pos4_hbm, out_hbm, idx_all, rows, pos4,
             gsem, ssem):
    wid = lax.axis_index("s") * NC + lax.axis_index("c")
    bat0 = wid * (ROWS_PER_W // L)

    pltpu.sync_copy(left_hbm.at[pl.ds(wid * IPW, IPW)], idx_all)
    pltpu.sync_copy(pos4_hbm, pos4)

    @pl.loop(0, NGRP)
    def _group(g):
        descs = []
        for b in range(NBUF):
            c = g * NBUF + b

            @pl.when(g > 0)
            def _():
                for k in range(BPC):
                    pltpu.make_async_copy(
                        rows.at[b, pl.ds(k * L, L)],
                        out_hbm.at[bat0 + k],
                        ssem.at[b],
                    ).wait()

            descs.append((
                pltpu.async_copy(table_hbm.at[idx_all.at[2 * c]],
                                 rows.at[b, pl.ds(0, GSZ)], gsem.at[b]),
                pltpu.async_copy(table_hbm.at[idx_all.at[2 * c + 1]],
                                 rows.at[b, pl.ds(GSZ, GSZ)], gsem.at[b]),
            ))

        for b in range(NBUF):
            c = g * NBUF + b
            for d in descs[b]:
                d.wait()

            @pl.loop(0, CHUNK)
            def _row(r):
                for q in range(D // 16):
                    x = pos4[r, pl.ds(q * 16, 16)]
                    plsc.addupdate(rows.at[b, r, pl.ds(q * 16, 16)], x)

            for k in range(BPC):
                pltpu.async_copy(rows.at[b, pl.ds(k * L, L)],
                                 out_hbm.at[c * BPC + bat0 + k], ssem.at[b])

    for b in range(NBUF):
        for k in range(BPC):
            pltpu.make_async_copy(rows.at[b, pl.ds(k * L, L)],
                                  out_hbm.at[bat0 + k], ssem.at[b]).wait()


@jax.jit
def _sc_gather(left2d, emb_left, pos4):
    mesh = plsc.VectorSubcoreMesh(core_axis_name="c", subcore_axis_name="s",
                                  num_cores=NC, num_subcores=NS)
    return pl.kernel(
        _sc_body,
        out_type=jax.ShapeDtypeStruct((B, L, D), jnp.float32),
        mesh=mesh,
        compiler_params=pltpu.CompilerParams(use_tc_tiling_on_sc=False),
        scratch_types=[
            pltpu.VMEM((IPW, GSZ), jnp.int32),
            pltpu.VMEM((NBUF, CHUNK, D), jnp.float32),
            pltpu.VMEM((CHUNK, D), jnp.float32),
            pltpu.SemaphoreType.DMA((NBUF,)),
            pltpu.SemaphoreType.DMA((NBUF,)),
        ],
    )(left2d, emb_left, pos4)


BC = 8192
NPAD = V + (8 - V % 8)


def _trans_body(in_ref, out_ref):
    x = in_ref[...]
    out_ref[:, 0:D] = jnp.swapaxes(x, 0, 1)


@jax.jit
def _detile(emb_t):
    grid = (NPAD + BC - 1) // BC
    return pl.pallas_call(
        _trans_body,
        grid=(grid,),
        in_specs=[pl.BlockSpec((64, BC), lambda g: (0, g))],
        out_specs=pl.BlockSpec((BC, 2 * D), lambda g: (g, 0)),
        out_shape=jax.ShapeDtypeStruct((NPAD, 2 * D), jnp.float32),
    )(emb_t)


def _mask_body(len_ref, out_ref):
    lens = len_ref[...]
    iota = lax.broadcasted_iota(jnp.int32, (B, L), 1)
    out_ref[...] = iota < lens


@jax.jit
def _mask_call(length):
    return pl.pallas_call(
        _mask_body,
        out_shape=jax.ShapeDtypeStruct((B, L), jnp.bool_),
    )(length)


def kernel(left, length, emb_left, pos_emb):
    left2d = left.reshape(BL // GSZ, GSZ) * 2
    pos4 = jnp.tile(pos_emb[:L], (CHUNK // L, 1))
    table_pad = _detile(emb_left.T).reshape(-1, D)
    seq = _sc_gather(left2d, table_pad, pos4)
    mask = _mask_call(length)
    return seq, mask


# speedup vs baseline: 1.8962x; 1.0424x over previous
import jax
import jax.numpy as jnp
from jax import lax
from jax.experimental import pallas as pl
from jax.experimental.pallas import tpu as pltpu
from jax.experimental.pallas import tpu_sc as plsc

B = 4096
L = 50
D = 64
V = 1000002
BL = B * L
NC = 2
NS = 16
NW = NC * NS
ROWS_PER_W = BL // NW
GSZ = 100
CHUNK = 2 * GSZ
BPC = CHUNK // L
CPW = ROWS_PER_W // CHUNK
IPW = ROWS_PER_W // GSZ
NBUF = 4
NGRP = CPW // NBUF


def _sc_body(left_hbm, table_hbm, pos4_hbm, out_hbm, idx_all, rows, pos4,
             gsem, ssem):
    wid = lax.axis_index("s") * NC + lax.axis_index("c")
    bat0 = wid * (ROWS_PER_W // L)

    pltpu.sync_copy(left_hbm.at[pl.ds(wid * IPW, IPW)], idx_all)
    pltpu.sync_copy(pos4_hbm, pos4)

    @pl.loop(0, NGRP)
    def _group(g):
        descs = []
        for b in range(NBUF):
            c = g * NBUF + b

            @pl.when(g > 0)
            def _():
                for k in range(BPC):
                    pltpu.make_async_copy(
                        rows.at[b, pl.ds(k * L, L)],
                        out_hbm.at[bat0 + k],
                        ssem.at[b],
                    ).wait()

            descs.append((
                pltpu.async_copy(table_hbm.at[idx_all.at[2 * c]],
                                 rows.at[b, pl.ds(0, GSZ)], gsem.at[b]),
                pltpu.async_copy(table_hbm.at[idx_all.at[2 * c + 1]],
                                 rows.at[b, pl.ds(GSZ, GSZ)], gsem.at[b]),
            ))

        for b in range(NBUF):
            c = g * NBUF + b
            for d in descs[b]:
                d.wait()

            @pl.loop(0, CHUNK)
            def _row(r):
                for q in range(D // 16):
                    x = pos4[r, pl.ds(q * 16, 16)]
                    plsc.addupdate(rows.at[b, r, pl.ds(q * 16, 16)], x)

            for k in range(BPC):
                pltpu.async_copy(rows.at[b, pl.ds(k * L, L)],
                                 out_hbm.at[c * BPC + bat0 + k], ssem.at[b])

    for b in range(NBUF):
        for k in range(BPC):
            pltpu.make_async_copy(rows.at[b, pl.ds(k * L, L)],
                                  out_hbm.at[bat0 + k], ssem.at[b]).wait()


@jax.jit
def _sc_gather(left2d, emb_left, pos4):
    mesh = plsc.VectorSubcoreMesh(core_axis_name="c", subcore_axis_name="s",
                                  num_cores=NC, num_subcores=NS)
    return pl.kernel(
        _sc_body,
        out_type=jax.ShapeDtypeStruct((B, L, D), jnp.float32),
        mesh=mesh,
        compiler_params=pltpu.CompilerParams(use_tc_tiling_on_sc=False),
        scratch_types=[
            pltpu.VMEM((IPW, GSZ), jnp.int32),
            pltpu.VMEM((NBUF, CHUNK, D), jnp.float32),
            pltpu.VMEM((CHUNK, D), jnp.float32),
            pltpu.SemaphoreType.DMA((NBUF,)),
            pltpu.SemaphoreType.DMA((NBUF,)),
        ],
    )(left2d, emb_left, pos4)


BC = 16384
NPAD = V + (8 - V % 8)


def _trans_body(in_ref, out_ref):
    x = in_ref[...]
    out_ref[:, 0:D] = jnp.swapaxes(x, 0, 1)


@jax.jit
def _detile(emb_t):
    grid = (NPAD + BC - 1) // BC
    return pl.pallas_call(
        _trans_body,
        grid=(grid,),
        in_specs=[pl.BlockSpec((64, BC), lambda g: (0, g))],
        out_specs=pl.BlockSpec((BC, 2 * D), lambda g: (g, 0)),
        out_shape=jax.ShapeDtypeStruct((NPAD, 2 * D), jnp.float32),
    )(emb_t)


def _mask_body(len_ref, out_ref):
    lens = len_ref[...]
    iota = lax.broadcasted_iota(jnp.int32, (B, L), 1)
    out_ref[...] = iota < lens


@jax.jit
def _mask_call(length):
    return pl.pallas_call(
        _mask_body,
        out_shape=jax.ShapeDtypeStruct((B, L), jnp.bool_),
    )(length)


def kernel(left, length, emb_left, pos_emb):
    left2d = left.reshape(BL // GSZ, GSZ) * 2
    pos4 = jnp.tile(pos_emb[:L], (CHUNK // L, 1))
    table_pad = _detile(emb_left.T).reshape(-1, D)
    seq = _sc_gather(left2d, table_pad, pos4)
    mask = _mask_call(length)
    return seq, mask
